# baseline (speedup 1.0000x reference)
import functools
import math

import jax
import jax.numpy as jnp
from jax import lax
from jax.experimental import pallas as pl
from jax.experimental.pallas import tpu as pltpu
from jax.experimental.pallas import tpu_sc as plsc

_NUM_CODES = 1000000
_CODE_DIM = 16
_EMBED_DIM = 32
_HIDDEN = 128
_OUT = 64
_B = 4096
_T = 20

_N_IDX = _B * _T * _CODE_DIM
_ROWS = _B * _T
_FEAT = _CODE_DIM * _EMBED_DIM
_NPART = 4
_XROWS = _NPART * _ROWS

_INFO = plsc.get_sparse_core_info()
_NC = _INFO.num_cores
_NS = _INFO.num_subcores
_NW = _NC * _NS
_BPW = _B // _NW
_LR_W = _ROWS // _NW
_CB = 4
_CHUNK = _CB * _T * _CODE_DIM
_STREAMS = _CHUNK // 128
_CLR = _CB * _T
_OUTER = _BPW // _CB


def _sc_gather(codes, table):
    mesh = plsc.VectorSubcoreMesh(core_axis_name="c", subcore_axis_name="s")

    @functools.partial(
        pl.kernel,
        mesh=mesh,
        out_type=jax.ShapeDtypeStruct((_XROWS, 128), jnp.bfloat16),
        scratch_types=[
            pltpu.VMEM((_CB, _T * _CODE_DIM), jnp.int32),
            pltpu.VMEM((_STREAMS, 128), jnp.int32),
            pltpu.VMEM((_CHUNK, _EMBED_DIM), jnp.bfloat16),
            pltpu.VMEM((_NPART * _CLR, 128), jnp.bfloat16),
            pltpu.SemaphoreType.DMA,
        ],
        compiler_params=pltpu.CompilerParams(use_tc_tiling_on_sc=False),
    )
    def k(codes_hbm, table_hbm, out_hbm, cidx_v, idx_v, rows_v, parts_v, sem):
        wid = lax.axis_index("s") * _NC + lax.axis_index("c")

        def body(outer, carry):
            b0 = wid * _BPW + outer * _CB
            lr0 = wid * _LR_W + outer * _CLR
            pltpu.sync_copy(codes_hbm.at[pl.ds(b0, _CB)], cidx_v)

            for b in range(_CB):
                for t in range(_T):
                    flat = (b * _T + t) * _CODE_DIM
                    idx_v[flat // 128, pl.ds(flat % 128, _CODE_DIM)] = (
                        cidx_v[b, pl.ds(t * _CODE_DIM, _CODE_DIM)]
                    )

            cps = []
            for j in range(_STREAMS):
                cp = pltpu.async_copy(
                    table_hbm.at[idx_v.at[j]],
                    rows_v.at[pl.ds(j * 128, 128)],
                    sem,
                )
                cps.append(cp)
            for cp in cps:
                cp.wait()

            def rbody(i, c2):
                vals = []
                for q in range(_NPART):
                    for c in range(4):
                        vals.append(rows_v[i * 16 + 4 * q + c, :])
                vi = 0
                for q in range(_NPART):
                    for c in range(4):
                        parts_v[q * _CLR + i, pl.ds(c * 32, 32)] = vals[vi]
                        vi += 1
                return c2

            lax.fori_loop(0, _CLR, rbody, 0)

            wcps = []
            for q in range(_NPART):
                wcps.append(pltpu.async_copy(
                    parts_v.at[pl.ds(q * _CLR, _CLR)],
                    out_hbm.at[pl.ds(q * _ROWS + lr0, _CLR)],
                    sem,
                ))
            for cp in wcps:
                cp.wait()
            return carry

        lax.fori_loop(0, _OUTER, body, 0)

    return k(codes, table)


_ROW_BLK = 1024
_PBLK = _ROWS // _ROW_BLK


def _mlp_body(x0_ref, x1_ref, x2_ref, x3_ref, w1_ref, b1_ref, gamma_ref,
              beta_ref, w2_ref, b2_ref, o_ref):
    h = b1_ref[...]
    for q, xq_ref in enumerate((x0_ref, x1_ref, x2_ref, x3_ref)):
        h = h + jnp.dot(
            xq_ref[...],
            w1_ref[pl.ds(q * 128, 128), :],
            preferred_element_type=jnp.float32,
        )
    h = h.astype(jnp.float32)
    h = 0.5 * h * (1.0 + lax.erf(h * (1.0 / math.sqrt(2.0))))
    mu = jnp.mean(h, axis=-1, keepdims=True)
    var = jnp.mean((h - mu) ** 2, axis=-1, keepdims=True)
    h = (h - mu) * lax.rsqrt(var + 1e-5)
    h = h * gamma_ref[...] + beta_ref[...]
    o_ref[...] = jnp.dot(h, w2_ref[...], preferred_element_type=jnp.float32) + b2_ref[...]


def _part_spec(q):
    return pl.BlockSpec((_ROW_BLK, 128), lambda i, q=q: (q * _PBLK + i, 0))


def _tc_mlp(x128, W1, b1, gamma, beta, W2, b2):
    grid = (_PBLK,)
    return pl.pallas_call(
        _mlp_body,
        grid=grid,
        in_specs=[
            _part_spec(0),
            _part_spec(1),
            _part_spec(2),
            _part_spec(3),
            pl.BlockSpec((_FEAT, _HIDDEN), lambda i: (0, 0)),
            pl.BlockSpec((1, _HIDDEN), lambda i: (0, 0)),
            pl.BlockSpec((1, _HIDDEN), lambda i: (0, 0)),
            pl.BlockSpec((1, _HIDDEN), lambda i: (0, 0)),
            pl.BlockSpec((_HIDDEN, _OUT), lambda i: (0, 0)),
            pl.BlockSpec((1, _OUT), lambda i: (0, 0)),
        ],
        out_specs=pl.BlockSpec((_ROW_BLK, _OUT), lambda i: (i, 0)),
        out_shape=jax.ShapeDtypeStruct((_ROWS, _OUT), jnp.float32),
    )(x128, x128, x128, x128, W1, b1, gamma, beta, W2, b2)


def kernel(codes, table, W1, b1, gamma, beta, W2, b2):
    x128 = _sc_gather(codes.reshape(_B, _T * _CODE_DIM), table.astype(jnp.bfloat16))
    out2d = _tc_mlp(
        x128,
        W1.astype(jnp.bfloat16),
        b1.reshape(1, _HIDDEN),
        gamma.reshape(1, _HIDDEN),
        beta.reshape(1, _HIDDEN),
        W2,
        b2.reshape(1, _OUT),
    )
    return out2d.reshape(_B, _T, _OUT)


# speedup vs baseline: 1.4650x; 1.4650x over previous
import functools
import math

import jax
import jax.numpy as jnp
from jax import lax
from jax.experimental import pallas as pl
from jax.experimental.pallas import tpu as pltpu
from jax.experimental.pallas import tpu_sc as plsc

_NUM_CODES = 1000000
_CODE_DIM = 16
_EMBED_DIM = 32
_HIDDEN = 128
_OUT = 64
_B = 4096
_T = 20

_N_IDX = _B * _T * _CODE_DIM
_ROWS = _B * _T
_FEAT = _CODE_DIM * _EMBED_DIM
_NPART = 4
_XROWS = _NPART * _ROWS

_INFO = plsc.get_sparse_core_info()
_NC = _INFO.num_cores
_NS = _INFO.num_subcores
_NW = _NC * _NS
_BPW = _B // _NW
_LR_W = _ROWS // _NW
_CB = 4
_CHUNK = _CB * _T * _CODE_DIM
_STREAMS = _CHUNK // 128
_CLR = _CB * _T
_OUTER = _BPW // _CB


def _sc_gather(codes, table):
    mesh = plsc.VectorSubcoreMesh(core_axis_name="c", subcore_axis_name="s")

    @functools.partial(
        pl.kernel,
        mesh=mesh,
        out_type=jax.ShapeDtypeStruct((_XROWS, 128), jnp.float32),
        scratch_types=[
            pltpu.VMEM((_CB, _T * _CODE_DIM), jnp.int32),
            pltpu.VMEM((_STREAMS, 128), jnp.int32),
            pltpu.VMEM((_CHUNK, _EMBED_DIM), jnp.float32),
            pltpu.VMEM((_NPART * _CLR, 128), jnp.float32),
            pltpu.SemaphoreType.DMA,
        ],
        compiler_params=pltpu.CompilerParams(use_tc_tiling_on_sc=False),
    )
    def k(codes_hbm, table_hbm, out_hbm, cidx_v, idx_v, rows_v, parts_v, sem):
        wid = lax.axis_index("s") * _NC + lax.axis_index("c")

        def body(outer, carry):
            b0 = wid * _BPW + outer * _CB
            lr0 = wid * _LR_W + outer * _CLR
            pltpu.sync_copy(codes_hbm.at[pl.ds(b0, _CB)], cidx_v)

            for b in range(_CB):
                for t in range(_T):
                    flat = (b * _T + t) * _CODE_DIM
                    idx_v[flat // 128, pl.ds(flat % 128, _CODE_DIM)] = (
                        cidx_v[b, pl.ds(t * _CODE_DIM, _CODE_DIM)]
                    )

            cps = []
            for j in range(_STREAMS):
                cp = pltpu.async_copy(
                    table_hbm.at[idx_v.at[j]],
                    rows_v.at[pl.ds(j * 128, 128)],
                    sem,
                )
                cps.append(cp)
            for cp in cps:
                cp.wait()

            def rbody(i, c2):
                vals = []
                for q in range(_NPART):
                    for c in range(4):
                        for half in range(2):
                            vals.append(
                                rows_v[i * 16 + 4 * q + c, pl.ds(half * 16, 16)]
                            )
                vi = 0
                for q in range(_NPART):
                    for c in range(4):
                        for half in range(2):
                            parts_v[q * _CLR + i, pl.ds(c * 32 + half * 16, 16)] = (
                                vals[vi]
                            )
                            vi += 1
                return c2

            lax.fori_loop(0, _CLR, rbody, 0)

            wcps = []
            for q in range(_NPART):
                wcps.append(pltpu.async_copy(
                    parts_v.at[pl.ds(q * _CLR, _CLR)],
                    out_hbm.at[pl.ds(q * _ROWS + lr0, _CLR)],
                    sem,
                ))
            for cp in wcps:
                cp.wait()
            return carry

        lax.fori_loop(0, _OUTER, body, 0)

    return k(codes, table)


_ROW_BLK = 2560
_PBLK = _ROWS // _ROW_BLK


def _mlp_body(x0_ref, x1_ref, x2_ref, x3_ref, w1_ref, b1_ref, gamma_ref,
              beta_ref, w2_ref, b2_ref, o_ref):
    h = b1_ref[...]
    for q, xq_ref in enumerate((x0_ref, x1_ref, x2_ref, x3_ref)):
        h = h + jnp.dot(
            xq_ref[...],
            w1_ref[pl.ds(q * 128, 128), :],
            preferred_element_type=jnp.float32,
        )
    h = 0.5 * h * (1.0 + lax.erf(h * (1.0 / math.sqrt(2.0))))
    mu = jnp.mean(h, axis=-1, keepdims=True)
    var = jnp.mean((h - mu) ** 2, axis=-1, keepdims=True)
    h = (h - mu) * lax.rsqrt(var + 1e-5)
    h = h * gamma_ref[...] + beta_ref[...]
    out = jnp.dot(h, w2_ref[...], preferred_element_type=jnp.float32) + b2_ref[...]
    o_ref[...] = out.reshape(_ROW_BLK // _T, _T, _OUT)


def _part_spec(q):
    return pl.BlockSpec((_ROW_BLK, 128), lambda i, q=q: (q * _PBLK + i, 0))


def _tc_mlp(x128, W1, b1, gamma, beta, W2, b2):
    grid = (_PBLK,)
    return pl.pallas_call(
        _mlp_body,
        grid=grid,
        in_specs=[
            _part_spec(0),
            _part_spec(1),
            _part_spec(2),
            _part_spec(3),
            pl.BlockSpec((_FEAT, _HIDDEN), lambda i: (0, 0)),
            pl.BlockSpec((1, _HIDDEN), lambda i: (0, 0)),
            pl.BlockSpec((1, _HIDDEN), lambda i: (0, 0)),
            pl.BlockSpec((1, _HIDDEN), lambda i: (0, 0)),
            pl.BlockSpec((_HIDDEN, _OUT), lambda i: (0, 0)),
            pl.BlockSpec((1, _OUT), lambda i: (0, 0)),
        ],
        out_specs=pl.BlockSpec((_ROW_BLK // _T, _T, _OUT), lambda i: (i, 0, 0)),
        out_shape=jax.ShapeDtypeStruct((_B, _T, _OUT), jnp.float32),
    )(x128, x128, x128, x128, W1, b1, gamma, beta, W2, b2)


def kernel(codes, table, W1, b1, gamma, beta, W2, b2):
    x128 = _sc_gather(codes.reshape(_B, _T * _CODE_DIM), table)
    return _tc_mlp(
        x128,
        W1,
        b1.reshape(1, _HIDDEN),
        gamma.reshape(1, _HIDDEN),
        beta.reshape(1, _HIDDEN),
        W2,
        b2.reshape(1, _OUT),
    )


# speedup vs baseline: 1.6454x; 1.1231x over previous
import functools
import math

import jax
import jax.numpy as jnp
from jax import lax
from jax.experimental import pallas as pl
from jax.experimental.pallas import tpu as pltpu
from jax.experimental.pallas import tpu_sc as plsc

_NUM_CODES = 1000000
_CODE_DIM = 16
_EMBED_DIM = 32
_HIDDEN = 128
_OUT = 64
_B = 4096
_T = 20

_N_IDX = _B * _T * _CODE_DIM
_ROWS = _B * _T
_FEAT = _CODE_DIM * _EMBED_DIM
_NPART = 4
_XROWS = _NPART * _ROWS

_INFO = plsc.get_sparse_core_info()
_NC = _INFO.num_cores
_NS = _INFO.num_subcores
_NW = _NC * _NS
_BPW = _B // _NW
_LR_W = _ROWS // _NW
_CB = 2
_CHUNK = _CB * _T * _CODE_DIM
_STREAMS = _CHUNK // 128
_CLR = _CB * _T
_OUTER = _BPW // _CB


def _sc_gather(codes, table):
    mesh = plsc.VectorSubcoreMesh(core_axis_name="c", subcore_axis_name="s")

    @functools.partial(
        pl.kernel,
        mesh=mesh,
        out_type=jax.ShapeDtypeStruct((_XROWS, 128), jnp.float32),
        scratch_types=[
            pltpu.VMEM((_CB, _T * _CODE_DIM), jnp.int32),
            pltpu.VMEM((_CB, _T * _CODE_DIM), jnp.int32),
            pltpu.VMEM((_STREAMS, 128), jnp.int32),
            pltpu.VMEM((_STREAMS, 128), jnp.int32),
            pltpu.VMEM((_CHUNK, _EMBED_DIM), jnp.float32),
            pltpu.VMEM((_CHUNK, _EMBED_DIM), jnp.float32),
            pltpu.VMEM((_NPART * _CLR, 128), jnp.float32),
            pltpu.VMEM((_NPART * _CLR, 128), jnp.float32),
            pltpu.SemaphoreType.DMA,
            pltpu.SemaphoreType.DMA,
            pltpu.SemaphoreType.DMA,
            pltpu.SemaphoreType.DMA,
        ],
        compiler_params=pltpu.CompilerParams(use_tc_tiling_on_sc=False),
    )
    def k(codes_hbm, table_hbm, out_hbm, cidx_a, cidx_b, idx_a, idx_b,
          rows_a, rows_b, parts_a, parts_b, sem_a, sem_b, semw_a, semw_b):
        wid = lax.axis_index("s") * _NC + lax.axis_index("c")

        def load_pack_fire(c, cidx_v, idx_v, rows_v, sem):
            b0 = wid * _BPW + c * _CB
            pltpu.sync_copy(codes_hbm.at[pl.ds(b0, _CB)], cidx_v)
            for b in range(_CB):
                for t in range(_T):
                    flat = (b * _T + t) * _CODE_DIM
                    idx_v[flat // 128, pl.ds(flat % 128, _CODE_DIM)] = (
                        cidx_v[b, pl.ds(t * _CODE_DIM, _CODE_DIM)]
                    )
            for j in range(_STREAMS):
                pltpu.async_copy(
                    table_hbm.at[idx_v.at[j]],
                    rows_v.at[pl.ds(j * 128, 128)],
                    sem,
                )

        def drain_streams(idx_v, rows_v, sem):
            for j in range(_STREAMS):
                pltpu.make_async_copy(
                    table_hbm.at[idx_v.at[j]],
                    rows_v.at[pl.ds(j * 128, 128)],
                    sem,
                ).wait()

        def repack_fire_wb(c, rows_v, parts_v, semw):
            lr0 = wid * _LR_W + c * _CLR

            def rbody(i, c2):
                vals = []
                for q in range(_NPART):
                    for cc in range(4):
                        for half in range(2):
                            vals.append(
                                rows_v[i * 16 + 4 * q + cc, pl.ds(half * 16, 16)]
                            )
                vi = 0
                for q in range(_NPART):
                    for cc in range(4):
                        for half in range(2):
                            parts_v[q * _CLR + i, pl.ds(cc * 32 + half * 16, 16)] = (
                                vals[vi]
                            )
                            vi += 1
                return c2

            lax.fori_loop(0, _CLR, rbody, 0)
            for q in range(_NPART):
                pltpu.async_copy(
                    parts_v.at[pl.ds(q * _CLR, _CLR)],
                    out_hbm.at[pl.ds(q * _ROWS + lr0, _CLR)],
                    semw,
                )

        def drain_wb(parts_v, semw):
            for q in range(_NPART):
                pltpu.make_async_copy(
                    parts_v.at[pl.ds(q * _CLR, _CLR)],
                    out_hbm.at[pl.ds(q * _ROWS, _CLR)],
                    semw,
                ).wait()

        npairs = _OUTER // 2
        load_pack_fire(0, cidx_a, idx_a, rows_a, sem_a)

        def body(k2, carry):
            c0 = 2 * k2
            c1 = c0 + 1
            load_pack_fire(c1, cidx_b, idx_b, rows_b, sem_b)
            drain_streams(idx_a, rows_a, sem_a)

            @pl.when(k2 > 0)
            def _():
                drain_wb(parts_a, semw_a)

            repack_fire_wb(c0, rows_a, parts_a, semw_a)

            @pl.when(k2 < npairs - 1)
            def _():
                load_pack_fire(c0 + 2, cidx_a, idx_a, rows_a, sem_a)

            drain_streams(idx_b, rows_b, sem_b)

            @pl.when(k2 > 0)
            def _():
                drain_wb(parts_b, semw_b)

            repack_fire_wb(c1, rows_b, parts_b, semw_b)
            return carry

        lax.fori_loop(0, npairs, body, 0)
        drain_wb(parts_a, semw_a)
        drain_wb(parts_b, semw_b)

    return k(codes, table)


_ROW_BLK = 2560
_PBLK = _ROWS // _ROW_BLK


def _mlp_body(x0_ref, x1_ref, x2_ref, x3_ref, w1_ref, b1_ref, gamma_ref,
              beta_ref, w2_ref, b2_ref, o_ref):
    h = b1_ref[...]
    for q, xq_ref in enumerate((x0_ref, x1_ref, x2_ref, x3_ref)):
        h = h + jnp.dot(
            xq_ref[...],
            w1_ref[pl.ds(q * 128, 128), :],
            preferred_element_type=jnp.float32,
        )
    h = 0.5 * h * (1.0 + lax.erf(h * (1.0 / math.sqrt(2.0))))
    mu = jnp.mean(h, axis=-1, keepdims=True)
    var = jnp.mean((h - mu) ** 2, axis=-1, keepdims=True)
    h = (h - mu) * lax.rsqrt(var + 1e-5)
    h = h * gamma_ref[...] + beta_ref[...]
    out = jnp.dot(h, w2_ref[...], preferred_element_type=jnp.float32) + b2_ref[...]
    o_ref[...] = out.reshape(_ROW_BLK // _T, _T, _OUT)


def _part_spec(q):
    return pl.BlockSpec((_ROW_BLK, 128), lambda i, q=q: (q * _PBLK + i, 0))


def _tc_mlp(x128, W1, b1, gamma, beta, W2, b2):
    grid = (_PBLK,)
    return pl.pallas_call(
        _mlp_body,
        grid=grid,
        in_specs=[
            _part_spec(0),
            _part_spec(1),
            _part_spec(2),
            _part_spec(3),
            pl.BlockSpec((_FEAT, _HIDDEN), lambda i: (0, 0)),
            pl.BlockSpec((1, _HIDDEN), lambda i: (0, 0)),
            pl.BlockSpec((1, _HIDDEN), lambda i: (0, 0)),
            pl.BlockSpec((1, _HIDDEN), lambda i: (0, 0)),
            pl.BlockSpec((_HIDDEN, _OUT), lambda i: (0, 0)),
            pl.BlockSpec((1, _OUT), lambda i: (0, 0)),
        ],
        out_specs=pl.BlockSpec((_ROW_BLK // _T, _T, _OUT), lambda i: (i, 0, 0)),
        out_shape=jax.ShapeDtypeStruct((_B, _T, _OUT), jnp.float32),
    )(x128, x128, x128, x128, W1, b1, gamma, beta, W2, b2)


def kernel(codes, table, W1, b1, gamma, beta, W2, b2):
    x128 = _sc_gather(codes.reshape(_B, _T * _CODE_DIM), table)
    return _tc_mlp(
        x128,
        W1,
        b1.reshape(1, _HIDDEN),
        gamma.reshape(1, _HIDDEN),
        beta.reshape(1, _HIDDEN),
        W2,
        b2.reshape(1, _OUT),
    )


# speedup vs baseline: 1.6558x; 1.0063x over previous
import functools
import math

import jax
import jax.numpy as jnp
from jax import lax
from jax.experimental import pallas as pl
from jax.experimental.pallas import tpu as pltpu
from jax.experimental.pallas import tpu_sc as plsc

_NUM_CODES = 1000000
_CODE_DIM = 16
_EMBED_DIM = 32
_HIDDEN = 128
_OUT = 64
_B = 4096
_T = 20

_N_IDX = _B * _T * _CODE_DIM
_ROWS = _B * _T
_FEAT = _CODE_DIM * _EMBED_DIM
_NPART = 4
_XROWS = _NPART * _ROWS

_INFO = plsc.get_sparse_core_info()
_NC = _INFO.num_cores
_NS = _INFO.num_subcores
_NW = _NC * _NS
_BPW = _B // _NW
_LR_W = _ROWS // _NW
_CB = 2
_CHUNK = _CB * _T * _CODE_DIM
_STREAMS = _CHUNK // 128
_CLR = _CB * _T
_OUTER = _BPW // _CB


def _sc_gather(codes, table, nb):
    mesh = plsc.VectorSubcoreMesh(core_axis_name="c", subcore_axis_name="s")
    bpw = nb // _NW
    lr_w = bpw * _T
    rows_total = nb * _T
    outer = bpw // _CB

    @functools.partial(
        pl.kernel,
        mesh=mesh,
        out_type=jax.ShapeDtypeStruct((_NPART * rows_total, 128), jnp.float32),
        scratch_types=[
            pltpu.VMEM((_CB, _T * _CODE_DIM), jnp.int32),
            pltpu.VMEM((_CB, _T * _CODE_DIM), jnp.int32),
            pltpu.VMEM((_STREAMS, 128), jnp.int32),
            pltpu.VMEM((_STREAMS, 128), jnp.int32),
            pltpu.VMEM((_CHUNK, _EMBED_DIM), jnp.float32),
            pltpu.VMEM((_CHUNK, _EMBED_DIM), jnp.float32),
            pltpu.VMEM((_NPART * _CLR, 128), jnp.float32),
            pltpu.VMEM((_NPART * _CLR, 128), jnp.float32),
            pltpu.SemaphoreType.DMA,
            pltpu.SemaphoreType.DMA,
            pltpu.SemaphoreType.DMA,
            pltpu.SemaphoreType.DMA,
        ],
        compiler_params=pltpu.CompilerParams(use_tc_tiling_on_sc=False),
    )
    def k(codes_hbm, table_hbm, out_hbm, cidx_a, cidx_b, idx_a, idx_b,
          rows_a, rows_b, parts_a, parts_b, sem_a, sem_b, semw_a, semw_b):
        wid = lax.axis_index("s") * _NC + lax.axis_index("c")

        def load_pack_fire(c, cidx_v, idx_v, rows_v, sem):
            b0 = wid * bpw + c * _CB
            pltpu.sync_copy(codes_hbm.at[pl.ds(b0, _CB)], cidx_v)
            for b in range(_CB):
                for t in range(_T):
                    flat = (b * _T + t) * _CODE_DIM
                    idx_v[flat // 128, pl.ds(flat % 128, _CODE_DIM)] = (
                        cidx_v[b, pl.ds(t * _CODE_DIM, _CODE_DIM)]
                    )
            for j in range(_STREAMS):
                pltpu.async_copy(
                    table_hbm.at[idx_v.at[j]],
                    rows_v.at[pl.ds(j * 128, 128)],
                    sem,
                )

        def drain_streams(idx_v, rows_v, sem):
            for j in range(_STREAMS):
                pltpu.make_async_copy(
                    table_hbm.at[idx_v.at[j]],
                    rows_v.at[pl.ds(j * 128, 128)],
                    sem,
                ).wait()

        def repack_fire_wb(c, rows_v, parts_v, semw):
            lr0 = wid * lr_w + c * _CLR

            def rbody(i, c2):
                vals = []
                for q in range(_NPART):
                    for cc in range(4):
                        for half in range(2):
                            vals.append(
                                rows_v[i * 16 + 4 * q + cc, pl.ds(half * 16, 16)]
                            )
                vi = 0
                for q in range(_NPART):
                    for cc in range(4):
                        for half in range(2):
                            parts_v[q * _CLR + i, pl.ds(cc * 32 + half * 16, 16)] = (
                                vals[vi]
                            )
                            vi += 1
                return c2

            lax.fori_loop(0, _CLR, rbody, 0)
            for q in range(_NPART):
                pltpu.async_copy(
                    parts_v.at[pl.ds(q * _CLR, _CLR)],
                    out_hbm.at[pl.ds(q * rows_total + lr0, _CLR)],
                    semw,
                )

        def drain_wb(parts_v, semw):
            for q in range(_NPART):
                pltpu.make_async_copy(
                    parts_v.at[pl.ds(q * _CLR, _CLR)],
                    out_hbm.at[pl.ds(q * rows_total, _CLR)],
                    semw,
                ).wait()

        npairs = outer // 2
        load_pack_fire(0, cidx_a, idx_a, rows_a, sem_a)

        def body(k2, carry):
            c0 = 2 * k2
            c1 = c0 + 1
            load_pack_fire(c1, cidx_b, idx_b, rows_b, sem_b)
            drain_streams(idx_a, rows_a, sem_a)

            @pl.when(k2 > 0)
            def _():
                drain_wb(parts_a, semw_a)

            repack_fire_wb(c0, rows_a, parts_a, semw_a)

            @pl.when(k2 < npairs - 1)
            def _():
                load_pack_fire(c0 + 2, cidx_a, idx_a, rows_a, sem_a)

            drain_streams(idx_b, rows_b, sem_b)

            @pl.when(k2 > 0)
            def _():
                drain_wb(parts_b, semw_b)

            repack_fire_wb(c1, rows_b, parts_b, semw_b)
            return carry

        lax.fori_loop(0, npairs, body, 0)
        drain_wb(parts_a, semw_a)
        drain_wb(parts_b, semw_b)

    return k(codes, table)


_ROW_BLK = 2560


def _mlp_body(x0_ref, x1_ref, x2_ref, x3_ref, w1_ref, b1_ref, gamma_ref,
              beta_ref, w2_ref, b2_ref, o_ref):
    h = b1_ref[...]
    for q, xq_ref in enumerate((x0_ref, x1_ref, x2_ref, x3_ref)):
        h = h + jnp.dot(
            xq_ref[...],
            w1_ref[pl.ds(q * 128, 128), :],
            preferred_element_type=jnp.float32,
        )
    h = 0.5 * h * (1.0 + lax.erf(h * (1.0 / math.sqrt(2.0))))
    mu = jnp.mean(h, axis=-1, keepdims=True)
    var = jnp.mean((h - mu) ** 2, axis=-1, keepdims=True)
    h = (h - mu) * lax.rsqrt(var + 1e-5)
    h = h * gamma_ref[...] + beta_ref[...]
    out = jnp.dot(h, w2_ref[...], preferred_element_type=jnp.float32) + b2_ref[...]
    o_ref[...] = out.reshape(_ROW_BLK // _T, _T, _OUT)


def _part_spec(q, pblk):
    return pl.BlockSpec((_ROW_BLK, 128), lambda i, q=q: (q * pblk + i, 0))


def _tc_mlp(x128, W1, b1, gamma, beta, W2, b2, nb):
    pblk = (nb * _T) // _ROW_BLK
    grid = (pblk,)
    return pl.pallas_call(
        _mlp_body,
        grid=grid,
        in_specs=[
            _part_spec(0, pblk),
            _part_spec(1, pblk),
            _part_spec(2, pblk),
            _part_spec(3, pblk),
            pl.BlockSpec((_FEAT, _HIDDEN), lambda i: (0, 0)),
            pl.BlockSpec((1, _HIDDEN), lambda i: (0, 0)),
            pl.BlockSpec((1, _HIDDEN), lambda i: (0, 0)),
            pl.BlockSpec((1, _HIDDEN), lambda i: (0, 0)),
            pl.BlockSpec((_HIDDEN, _OUT), lambda i: (0, 0)),
            pl.BlockSpec((1, _OUT), lambda i: (0, 0)),
        ],
        out_specs=pl.BlockSpec((_ROW_BLK // _T, _T, _OUT), lambda i: (i, 0, 0)),
        out_shape=jax.ShapeDtypeStruct((nb, _T, _OUT), jnp.float32),
    )(x128, x128, x128, x128, W1, b1, gamma, beta, W2, b2)


def kernel(codes, table, W1, b1, gamma, beta, W2, b2):
    codes2 = codes.reshape(_B, _T * _CODE_DIM)
    nb = _B // 2
    args = (
        W1,
        b1.reshape(1, _HIDDEN),
        gamma.reshape(1, _HIDDEN),
        beta.reshape(1, _HIDDEN),
        W2,
        b2.reshape(1, _OUT),
    )
    xa = _sc_gather(codes2[:nb], table, nb)
    xb = _sc_gather(codes2[nb:], table, nb)
    oa = _tc_mlp(xa, *args, nb)
    ob = _tc_mlp(xb, *args, nb)
    return jnp.concatenate([oa, ob], axis=0)
